# baseline (speedup 1.0000x reference)
import functools

import jax
import jax.numpy as jnp
from jax import lax
from jax.experimental import pallas as pl
from jax.experimental.pallas import tpu as pltpu
from jax.experimental.pallas import tpu_sc as plsc

B = 16384
D = 32
H = 64
NC = 2
NS = 16
NW = NC * NS
BPW = B // NW

@functools.cache
def _make_sc_gather():
    mesh = plsc.VectorSubcoreMesh(core_axis_name="c", subcore_axis_name="s")

    @functools.partial(
        pl.kernel,
        mesh=mesh,
        out_type=jax.ShapeDtypeStruct((B, D), jnp.float32),
        scratch_types=[
            pltpu.VMEM((BPW,), jnp.int32),
            pltpu.VMEM((BPW, D), jnp.float32),
            pltpu.SemaphoreType.DMA,
        ],
        compiler_params=pltpu.CompilerParams(use_tc_tiling_on_sc=False),
    )
    def _sc_gather(idx_hbm, table_hbm, out_hbm, idx_v, rows_v, sem):
        wid = lax.axis_index("s") * NC + lax.axis_index("c")
        base = wid * BPW
        pltpu.sync_copy(idx_hbm.at[pl.ds(base, BPW)], idx_v)
        pltpu.async_copy(table_hbm.at[idx_v], rows_v, sem).wait()
        pltpu.sync_copy(rows_v, out_hbm.at[pl.ds(base, BPW)])

    return _sc_gather


def _mlp_body(uv_ref, bio_ref, mean_ref, var_ref, w1_ref, b1_ref, w2_ref,
              b2_ref, out_ref):
    inv = lax.rsqrt(var_ref[:] + 1e-7)
    xn = (bio_ref[:] - mean_ref[:]) * inv
    h = jnp.dot(xn, w1_ref[:], preferred_element_type=jnp.float32)
    h = jnp.maximum(h + b1_ref[:], 0.0)
    bio_vec = jnp.dot(h, w2_ref[:], preferred_element_type=jnp.float32)
    bio_vec = bio_vec + b2_ref[:]
    out_ref[:] = jnp.concatenate([uv_ref[:], bio_vec], axis=1)


_BLK = 2048


def _tc_mlp(user_vec, bio, mean2, var2, W1, b1_2, W2, b2_2):
    return pl.pallas_call(
        _mlp_body,
        grid=(B // _BLK,),
        in_specs=[
            pl.BlockSpec((_BLK, D), lambda i: (i, 0)),
            pl.BlockSpec((_BLK, 3), lambda i: (i, 0)),
            pl.BlockSpec((1, 3), lambda i: (0, 0)),
            pl.BlockSpec((1, 3), lambda i: (0, 0)),
            pl.BlockSpec((3, H), lambda i: (0, 0)),
            pl.BlockSpec((1, H), lambda i: (0, 0)),
            pl.BlockSpec((H, D), lambda i: (0, 0)),
            pl.BlockSpec((1, D), lambda i: (0, 0)),
        ],
        out_specs=pl.BlockSpec((_BLK, 2 * D), lambda i: (i, 0)),
        out_shape=jax.ShapeDtypeStruct((B, 2 * D), jnp.float32),
    )(user_vec, bio, mean2, var2, W1, b1_2, W2, b2_2)


def kernel(user_id, P, E, I, emb_table, norm_mean, norm_var, W1, b1, W2, b2):
    user_vec = _make_sc_gather()(user_id, emb_table)
    bio = jnp.stack([P, E, I], axis=1)
    return _tc_mlp(user_vec, bio,
                   norm_mean.reshape(1, 3), norm_var.reshape(1, 3),
                   W1, b1.reshape(1, H), W2, b2.reshape(1, D))
